# baseline (device time: 17349 ns/iter reference)
import jax
import jax.numpy as jnp
from jax import lax
from jax.experimental import pallas as pl
from jax.experimental.pallas import tpu as pltpu


def kernel(ids, E):
    T = ids.shape[0]
    V_local, D = E.shape

    ids2 = ids.reshape(T, 1)

    def body(ids_ref, e_ref, out_ref, pbuf, rbuf, send_sem, recv_sem):
        my_x = lax.axis_index("x")
        my_y = lax.axis_index("y")
        my_z = lax.axis_index("z")
        y_partner = (my_x, 1 - my_y, my_z)

        barrier = pltpu.get_barrier_semaphore()
        pl.semaphore_signal(
            barrier, inc=1, device_id=y_partner,
            device_id_type=pl.DeviceIdType.MESH,
        )
        pl.semaphore_wait(barrier, 1)

        local = ids_ref[:, :] - my_y * V_local
        iota = lax.broadcasted_iota(jnp.int32, (T, V_local), 1)
        onehot = (iota == local).astype(jnp.bfloat16)
        partial = jnp.dot(
            onehot, e_ref[:, :].astype(jnp.bfloat16),
            preferred_element_type=jnp.float32,
        )
        pbuf[:, :] = partial.astype(jnp.bfloat16)

        rdma = pltpu.make_async_remote_copy(
            src_ref=pbuf, dst_ref=rbuf,
            send_sem=send_sem, recv_sem=recv_sem,
            device_id=y_partner, device_id_type=pl.DeviceIdType.MESH,
        )
        rdma.start()
        rdma.wait()

        out_ref[:, :] = pbuf[:, :].astype(jnp.float32) + rbuf[:, :].astype(
            jnp.float32
        )

    return pl.pallas_call(
        body,
        out_shape=jax.ShapeDtypeStruct((T, D), jnp.float32),
        in_specs=[
            pl.BlockSpec(memory_space=pltpu.VMEM),
            pl.BlockSpec(memory_space=pltpu.VMEM),
        ],
        out_specs=pl.BlockSpec(memory_space=pltpu.VMEM),
        scratch_shapes=[
            pltpu.VMEM((T, D), jnp.bfloat16),
            pltpu.VMEM((T, D), jnp.bfloat16),
            pltpu.SemaphoreType.DMA,
            pltpu.SemaphoreType.DMA,
        ],
        compiler_params=pltpu.CompilerParams(collective_id=0),
    )(ids2, E)
